# baseline (speedup 1.0000x reference)
import jax
import jax.numpy as jnp
from jax import lax
from jax.experimental import pallas as pl
from jax.experimental.pallas import tpu as pltpu
from jax.experimental.pallas import tpu_sc as plsc

_DIM = 1 << 20
_BATCH = 32
_BLK = 1 << 14
_HALF = _BLK // 2
_NBLK = _DIM // _BLK
_VPB = _BLK // 16

_NC = 2
_NS = 16


def _body(x_hbm, out_hbm, in0, in1, out0, out1, sin0, sin1, sout0, sout1):
    wid = lax.axis_index("s") * _NC + lax.axis_index("c")
    x_row = x_hbm.at[wid]
    out_row = out_hbm.at[wid]
    row_off = 0
    ins = (in0, in1)
    outs = (out0, out1)
    sins = (sin0, sin1)
    souts = (sout0, sout1)

    lane = lax.broadcasted_iota(jnp.int32, (16,), 0)
    glane = lane ^ (lane >> 1)

    def start_in(h, b):
        src = h ^ (h >> 1)
        base = row_off + src * _BLK
        pltpu.async_copy(x_row.at[pl.ds(base, _BLK)], ins[b], sins[b])

    def wait_in(b):
        pltpu.make_async_copy(x_row.at[pl.ds(row_off, _BLK)], ins[b],
                              sins[b]).wait()

    def start_out(h, b):
        pltpu.async_copy(outs[b], out_row.at[pl.ds(row_off + h * _BLK, _BLK)],
                         souts[b])

    def wait_out(h, b):
        pltpu.make_async_copy(outs[b],
                              out_row.at[pl.ds(row_off + h * _BLK, _BLK)],
                              souts[b]).wait()

    def compute(b):
        @plsc.parallel_loop(0, _VPB, unroll=8)
        def _vec(k):
            idx = glane ^ ((((k * 2) ^ k) * 8) ^ (b * _HALF))
            outs[b][pl.ds(k * 16, 16)] = plsc.load_gather(ins[b], [idx])

    start_in(0, 0)
    start_in(1, 1)

    def pair_step(hh, carry):
        for b in range(2):
            h = hh * 2 + b
            wait_in(b)

            @pl.when(hh > 0)
            def _():
                wait_out(h - 2, b)

            compute(b)
            start_out(h, b)

            @pl.when(hh < _NBLK // 2 - 1)
            def _():
                start_in(h + 2, b)

        return carry

    lax.fori_loop(0, _NBLK // 2, pair_step, None)
    wait_out(_NBLK - 2, 0)
    wait_out(_NBLK - 1, 1)


_permute = pl.kernel(
    _body,
    out_type=jax.ShapeDtypeStruct((_BATCH, _DIM), jnp.float32),
    mesh=plsc.VectorSubcoreMesh(core_axis_name="c", subcore_axis_name="s"),
    scratch_types=[
        pltpu.VMEM((_BLK,), jnp.float32),
        pltpu.VMEM((_BLK,), jnp.float32),
        pltpu.VMEM((_BLK,), jnp.float32),
        pltpu.VMEM((_BLK,), jnp.float32),
        pltpu.SemaphoreType.DMA,
        pltpu.SemaphoreType.DMA,
        pltpu.SemaphoreType.DMA,
        pltpu.SemaphoreType.DMA,
    ],
    compiler_params=pltpu.CompilerParams(needs_layout_passes=False),
)


@jax.jit
def kernel(x, ind):
    del ind
    return _permute(x)


# speedup vs baseline: 1.0350x; 1.0350x over previous
import jax
import jax.numpy as jnp
from jax import lax
from jax.experimental import pallas as pl
from jax.experimental.pallas import tpu as pltpu
from jax.experimental.pallas import tpu_sc as plsc

_DIM = 1 << 20
_BATCH = 32
_LOGBLK = 13
_BLK = 1 << _LOGBLK
_HALF = _BLK // 2
_NBLK = _DIM // _BLK
_VPB = _BLK // 16
_NBUF = 4

_NC = 2
_NS = 16


def _body(x_hbm, out_hbm, *refs):
    ins = refs[0:_NBUF]
    outs = refs[_NBUF:2 * _NBUF]
    sins = refs[2 * _NBUF:3 * _NBUF]
    souts = refs[3 * _NBUF:4 * _NBUF]

    wid = lax.axis_index("s") * _NC + lax.axis_index("c")
    x_row = x_hbm.at[wid]
    out_row = out_hbm.at[wid]

    lane = lax.broadcasted_iota(jnp.int32, (16,), 0)
    glane = lane ^ (lane >> 1)

    def start_in(h, b):
        src = h ^ (h >> 1)
        base = src * _BLK
        par = b & 1
        pltpu.async_copy(
            x_row.at[pl.ds(base, _HALF)],
            ins[b].at[pl.ds(par * _HALF, _HALF)], sins[b])
        pltpu.async_copy(
            x_row.at[pl.ds(base + _HALF, _HALF)],
            ins[b].at[pl.ds((1 - par) * _HALF, _HALF)], sins[b])

    def wait_in(b):
        pltpu.make_async_copy(x_row.at[pl.ds(0, _BLK)], ins[b], sins[b]).wait()

    def start_out(h, b):
        pltpu.async_copy(outs[b], out_row.at[pl.ds(h * _BLK, _BLK)], souts[b])

    def wait_out(h, b):
        pltpu.make_async_copy(outs[b], out_row.at[pl.ds(h * _BLK, _BLK)],
                              souts[b]).wait()

    def compute(b):
        @plsc.parallel_loop(0, _VPB, unroll=8)
        def _vec(k):
            idx = glane ^ (((k * 2) ^ k) * 8)
            outs[b][pl.ds(k * 16, 16)] = plsc.load_gather(ins[b], [idx])

    for b in range(_NBUF):
        start_in(b, b)

    def group_step(g, carry):
        for b in range(_NBUF):
            h = g * _NBUF + b
            wait_in(b)

            @pl.when(g > 0)
            def _():
                wait_out(h - _NBUF, b)

            compute(b)
            start_out(h, b)

            @pl.when(g < _NBLK // _NBUF - 1)
            def _():
                start_in(h + _NBUF, b)

        return carry

    lax.fori_loop(0, _NBLK // _NBUF, group_step, None)
    for b in range(_NBUF):
        wait_out(_NBLK - _NBUF + b, b)


_permute = pl.kernel(
    _body,
    out_type=jax.ShapeDtypeStruct((_BATCH, _DIM), jnp.float32),
    mesh=plsc.VectorSubcoreMesh(core_axis_name="c", subcore_axis_name="s"),
    scratch_types=(
        [pltpu.VMEM((_BLK,), jnp.float32) for _ in range(2 * _NBUF)]
        + [pltpu.SemaphoreType.DMA for _ in range(2 * _NBUF)]
    ),
    compiler_params=pltpu.CompilerParams(needs_layout_passes=False),
)


@jax.jit
def kernel(x, ind):
    del ind
    return _permute(x)
